# Initial kernel scaffold; baseline (speedup 1.0000x reference)
import jax
import jax.numpy as jnp
from jax.experimental import pallas as pl


def kernel(x, edge_index, batch, W1, b1, W2, b2, Wfc, bfc):
    raise NotImplementedError("write your pallas kernel here")


# speedup vs baseline: 20.7663x; 20.7663x over previous
import functools

import jax
import jax.numpy as jnp
from jax import lax
from jax.experimental import pallas as pl
from jax.experimental.pallas import tpu as pltpu
from jax.experimental.pallas import tpu_sc as plsc

NNODES = 10000
NEDGES = 320000
D = 128
NGRAPHS = 256

NC, NS = 2, 16
NW = NC * NS
ROWS = 10016
RPT = ROWS // NS
EPAD = 327680
CHUNK = 128
NCHUNK = EPAD // (NW * CHUNK)
DEGR = 640
RBLK = 2504

_mesh = plsc.VectorSubcoreMesh(
    core_axis_name="c", subcore_axis_name="s", num_cores=NC, num_subcores=NS)


def _deg_body(dst_hbm, ones_hbm, zero_hbm, out_hbm, dstix, ones_v, shared):
    c = lax.axis_index("c")
    s = lax.axis_index("s")
    wid = s * NC + c
    seg = DEGR * 16 // NS
    pltpu.sync_copy(dst_hbm.at[wid], dstix)
    pltpu.sync_copy(ones_hbm, ones_v)
    pltpu.sync_copy(zero_hbm, shared.at[pl.ds(s * seg, seg)])
    plsc.subcore_barrier()

    def chunk(j, carry):
        pltpu.sync_copy(ones_v, shared.at[dstix.at[j]], add=True)
        return carry

    lax.fori_loop(0, NCHUNK, chunk, 0)
    plsc.subcore_barrier()
    pltpu.sync_copy(shared.at[pl.ds(s * seg, seg)], out_hbm.at[c, s])


def _deg_call(dst3, onesA, zerosA):
    f = pl.kernel(
        _deg_body,
        out_type=jax.ShapeDtypeStruct((NC, NS, DEGR * 16 // NS, 16),
                                      jnp.float32),
        mesh=_mesh,
        scratch_types=[
            pltpu.VMEM((NCHUNK, CHUNK), jnp.int32),
            pltpu.VMEM((CHUNK, 16), jnp.float32),
            pltpu.VMEM_SHARED((DEGR * 16, 16), jnp.float32),
        ],
    )
    return f(dst3, onesA, zerosA)


def _scat_body(y_hbm, src_hbm, dst_hbm, zero_hbm, out_hbm,
               srcix, dstix, rowbuf, accum, sem):
    c = lax.axis_index("c")
    s = lax.axis_index("s")
    wid = s * NC + c
    pltpu.sync_copy(src_hbm.at[wid], srcix)
    pltpu.sync_copy(dst_hbm.at[wid], dstix)
    pltpu.sync_copy(zero_hbm, accum.at[pl.ds(s * RPT, RPT)])
    plsc.subcore_barrier()

    def chunk(j, carry):
        pltpu.async_copy(y_hbm.at[srcix.at[j]], rowbuf, sem).wait()
        pltpu.sync_copy(rowbuf, accum.at[dstix.at[j]], add=True)
        return carry

    lax.fori_loop(0, NCHUNK, chunk, 0)
    plsc.subcore_barrier()
    pltpu.sync_copy(accum.at[pl.ds(s * RPT, RPT)], out_hbm.at[c, s])


def _scat_call(y, src3, dst3, zerosC):
    f = pl.kernel(
        _scat_body,
        out_type=jax.ShapeDtypeStruct((NC, NS, RPT, D), jnp.float32),
        mesh=_mesh,
        scratch_types=[
            pltpu.VMEM((NCHUNK, CHUNK), jnp.int32),
            pltpu.VMEM((NCHUNK, CHUNK), jnp.int32),
            pltpu.VMEM((CHUNK, D), jnp.float32),
            pltpu.VMEM_SHARED((ROWS, D), jnp.float32),
            pltpu.SemaphoreType.DMA,
        ],
    )
    return f(y, src3, dst3, zerosC).reshape(NC, ROWS, D)


def _b_body(deg_ref, x_ref, w1_ref, y_ref, dinv_ref):
    deg = deg_ref[0] + deg_ref[1] + 1.0
    dinv = lax.rsqrt(deg)
    xw = jnp.dot(x_ref[...], w1_ref[...], preferred_element_type=jnp.float32)
    y_ref[...] = xw * dinv
    dinv_ref[...] = dinv


def _b_call(degp, x_p, W1):
    return pl.pallas_call(
        _b_body,
        grid=(ROWS // RBLK,),
        in_specs=[
            pl.BlockSpec((NC, RBLK, 1), lambda i: (0, i, 0)),
            pl.BlockSpec((RBLK, D), lambda i: (i, 0)),
            pl.BlockSpec((D, D), lambda i: (0, 0)),
        ],
        out_specs=[
            pl.BlockSpec((RBLK, D), lambda i: (i, 0)),
            pl.BlockSpec((RBLK, 1), lambda i: (i, 0)),
        ],
        out_shape=[
            jax.ShapeDtypeStruct((ROWS, D), jnp.float32),
            jax.ShapeDtypeStruct((ROWS, 1), jnp.float32),
        ],
    )(degp, x_p, W1)


def _d_body(sp_ref, y1_ref, dinv_ref, b1_ref, w2_ref, y2_ref):
    stot = sp_ref[0] + sp_ref[1] + y1_ref[...]
    dinv = dinv_ref[...]
    h = jnp.maximum(stot * dinv + b1_ref[...], 0.0)
    y2_ref[...] = jnp.dot(h, w2_ref[...],
                          preferred_element_type=jnp.float32) * dinv


def _d_call(s1, y1, dinv, b1, W2):
    return pl.pallas_call(
        _d_body,
        grid=(ROWS // RBLK,),
        in_specs=[
            pl.BlockSpec((NC, RBLK, D), lambda i: (0, i, 0)),
            pl.BlockSpec((RBLK, D), lambda i: (i, 0)),
            pl.BlockSpec((RBLK, 1), lambda i: (i, 0)),
            pl.BlockSpec((1, D), lambda i: (0, 0)),
            pl.BlockSpec((D, D), lambda i: (0, 0)),
        ],
        out_specs=pl.BlockSpec((RBLK, D), lambda i: (i, 0)),
        out_shape=jax.ShapeDtypeStruct((ROWS, D), jnp.float32),
    )(s1, y1, dinv, b1, W2)


def _e_body(sp_ref, y2_ref, dinv_ref, b2_ref, batch_ref, wfc_ref, bfc_ref,
            out_ref, sums, counts):
    i = pl.program_id(0)

    @pl.when(i == 0)
    def _():
        sums[...] = jnp.zeros_like(sums)
        counts[...] = jnp.zeros_like(counts)

    stot = sp_ref[0] + sp_ref[1] + y2_ref[...]
    h = jnp.maximum(stot * dinv_ref[...] + b2_ref[...], 0.0)
    ids = batch_ref[0]
    mask = (lax.broadcasted_iota(jnp.int32, (NGRAPHS, RBLK), 0)
            == ids).astype(jnp.float32)
    sums[...] += jnp.dot(mask, h, preferred_element_type=jnp.float32)
    counts[...] += jnp.sum(mask, axis=1, keepdims=True)

    @pl.when(i == ROWS // RBLK - 1)
    def _():
        g = sums[...] / jnp.maximum(counts[...], 1.0)
        z = jnp.dot(g, wfc_ref[...],
                    preferred_element_type=jnp.float32) + bfc_ref[...]
        out_ref[...] = jax.nn.sigmoid(z)


def _e_call(s2, y2, dinv, b2, batch_p, Wfc, bfc):
    return pl.pallas_call(
        _e_body,
        grid=(ROWS // RBLK,),
        in_specs=[
            pl.BlockSpec((NC, RBLK, D), lambda i: (0, i, 0)),
            pl.BlockSpec((RBLK, D), lambda i: (i, 0)),
            pl.BlockSpec((RBLK, 1), lambda i: (i, 0)),
            pl.BlockSpec((1, D), lambda i: (0, 0)),
            pl.BlockSpec((1, 1, RBLK), lambda i: (i, 0, 0)),
            pl.BlockSpec((D, 1), lambda i: (0, 0)),
            pl.BlockSpec((1, 1), lambda i: (0, 0)),
        ],
        out_specs=pl.BlockSpec((NGRAPHS, 1), lambda i: (0, 0)),
        out_shape=jax.ShapeDtypeStruct((NGRAPHS, 1), jnp.float32),
        scratch_shapes=[
            pltpu.VMEM((NGRAPHS, D), jnp.float32),
            pltpu.VMEM((NGRAPHS, 1), jnp.float32),
        ],
    )(s2, y2, dinv, b2, batch_p, Wfc, bfc)


def kernel(x, edge_index, batch, W1, b1, W2, b2, Wfc, bfc):
    npad = EPAD - NEDGES
    spread = jnp.arange(npad, dtype=jnp.int32) % 16
    src_p = jnp.concatenate([edge_index[0], spread])
    dst_p = jnp.concatenate([edge_index[1], NNODES + spread])
    src3 = src_p.reshape(NW, NCHUNK, CHUNK)
    dst3 = dst_p.reshape(NW, NCHUNK, CHUNK)
    x_p = jnp.concatenate([x, jnp.zeros((ROWS - NNODES, D), x.dtype)])
    batch_p = jnp.concatenate(
        [batch, jnp.full((ROWS - NNODES,), NGRAPHS, jnp.int32)]
    ).reshape(ROWS // RBLK, 1, RBLK)
    zerosA = jnp.zeros((DEGR * 16 // NS, 16), jnp.float32)
    onesA = jnp.ones((CHUNK, 16), jnp.float32)
    zerosC = jnp.zeros((RPT, D), jnp.float32)

    deg_parts = _deg_call(dst3, onesA, zerosA)
    degp = deg_parts.reshape(NC, DEGR * 16, 16)[:, :ROWS, :1]
    y1, dinv = _b_call(degp, x_p, W1)
    s1 = _scat_call(y1, src3, dst3, zerosC)
    y2 = _d_call(s1, y1, dinv, b1.reshape(1, D), W2)
    s2 = _scat_call(y2, src3, dst3, zerosC)
    out = _e_call(s2, y2, dinv, b2.reshape(1, D), batch_p, Wfc,
                  bfc.reshape(1, 1))
    return out


# speedup vs baseline: 28.7686x; 1.3854x over previous
import functools

import jax
import jax.numpy as jnp
from jax import lax
from jax.experimental import pallas as pl
from jax.experimental.pallas import tpu as pltpu
from jax.experimental.pallas import tpu_sc as plsc

NNODES = 10000
NEDGES = 320000
D = 128
NGRAPHS = 256

NC, NS = 2, 16
NW = NC * NS
ROWS = 10016
RPT = ROWS // NS
EPAD = 327680
CHUNK = 128
NCHUNK = EPAD // (NW * CHUNK)
DEGR = 640
RBLK = 2504

_mesh = plsc.VectorSubcoreMesh(
    core_axis_name="c", subcore_axis_name="s", num_cores=NC, num_subcores=NS)


def _deg_body(dst_hbm, ones_hbm, zero_hbm, out_hbm, dstix, ones_v, shared):
    c = lax.axis_index("c")
    s = lax.axis_index("s")
    wid = s * NC + c
    seg = DEGR * 16 // NS
    pltpu.sync_copy(dst_hbm.at[wid], dstix)
    pltpu.sync_copy(ones_hbm, ones_v)
    pltpu.sync_copy(zero_hbm, shared.at[pl.ds(s * seg, seg)])
    plsc.subcore_barrier()

    def chunk(j, carry):
        pltpu.sync_copy(ones_v, shared.at[dstix.at[j]], add=True)
        return carry

    lax.fori_loop(0, NCHUNK, chunk, 0)
    plsc.subcore_barrier()
    pltpu.sync_copy(shared.at[pl.ds(s * seg, seg)], out_hbm.at[c, s])


def _deg_call(dst3, onesA, zerosA):
    f = pl.kernel(
        _deg_body,
        out_type=jax.ShapeDtypeStruct((NC, NS, DEGR * 16 // NS, 16),
                                      jnp.float32),
        mesh=_mesh,
        scratch_types=[
            pltpu.VMEM((NCHUNK, CHUNK), jnp.int32),
            pltpu.VMEM((CHUNK, 16), jnp.float32),
            pltpu.VMEM_SHARED((DEGR * 16, 16), jnp.float32),
        ],
    )
    return f(dst3, onesA, zerosA)


def _scat_body(y_hbm, src_hbm, dst_hbm, zero_hbm, out_hbm,
               srcix, dstix, rowbuf, accum,
               rsem0, rsem1, isem0, isem1, isem2, isem3):
    c = lax.axis_index("c")
    s = lax.axis_index("s")
    wid = s * NC + c
    rsems = (rsem0, rsem1)
    isems = (isem0, isem1, isem2, isem3)
    pltpu.sync_copy(zero_hbm, accum.at[pl.ds(s * RPT, RPT)])
    plsc.subcore_barrier()

    def issue_idx(j, k):
        pltpu.async_copy(src_hbm.at[wid, j], srcix.at[k], isems[k])
        pltpu.async_copy(dst_hbm.at[wid, j], dstix.at[k], isems[k])

    def wait_idx(k):
        pltpu.make_async_copy(src_hbm.at[0, 0], srcix.at[k], isems[k]).wait()
        pltpu.make_async_copy(dst_hbm.at[0, 0], dstix.at[k], isems[k]).wait()

    for k in range(4):
        issue_idx(k, k)
    for b in range(2):
        wait_idx(b)
        pltpu.async_copy(y_hbm.at[srcix.at[b]], rowbuf.at[b], rsems[b])

    def outer(t, carry):
        for b in range(4):
            j = 4 * t + b
            rb = b % 2
            pltpu.make_async_copy(
                y_hbm.at[srcix.at[0]], rowbuf.at[rb], rsems[rb]).wait()
            pltpu.sync_copy(rowbuf.at[rb], accum.at[dstix.at[b]], add=True)
            issue_idx(jnp.minimum(j + 4, NCHUNK - 1), b)
            k2 = (b + 2) % 4
            wait_idx(k2)
            pltpu.async_copy(y_hbm.at[srcix.at[k2]], rowbuf.at[rb],
                             rsems[rb])
        return carry

    lax.fori_loop(0, NCHUNK // 4, outer, 0)
    for k in (2, 3):
        wait_idx(k)
    for b in range(2):
        pltpu.make_async_copy(
            y_hbm.at[srcix.at[0]], rowbuf.at[b], rsems[b]).wait()
    plsc.subcore_barrier()
    pltpu.sync_copy(accum.at[pl.ds(s * RPT, RPT)], out_hbm.at[c, s])


def _scat_call(y, src3, dst3, zerosC):
    f = pl.kernel(
        _scat_body,
        out_type=jax.ShapeDtypeStruct((NC, NS, RPT, D), jnp.float32),
        mesh=_mesh,
        scratch_types=[
            pltpu.VMEM((4, CHUNK), jnp.int32),
            pltpu.VMEM((4, CHUNK), jnp.int32),
            pltpu.VMEM((2, CHUNK, D), jnp.float32),
            pltpu.VMEM_SHARED((ROWS, D), jnp.float32),
            pltpu.SemaphoreType.DMA,
            pltpu.SemaphoreType.DMA,
            pltpu.SemaphoreType.DMA,
            pltpu.SemaphoreType.DMA,
            pltpu.SemaphoreType.DMA,
            pltpu.SemaphoreType.DMA,
        ],
    )
    return f(y, src3, dst3, zerosC).reshape(NC, ROWS, D)


def _b_body(deg_ref, x_ref, w1_ref, y_ref, dinv_ref):
    deg = deg_ref[0] + deg_ref[1] + 1.0
    dinv = lax.rsqrt(deg)
    xw = jnp.dot(x_ref[...], w1_ref[...], preferred_element_type=jnp.float32)
    y_ref[...] = xw * dinv
    dinv_ref[...] = dinv


def _b_call(degp, x_p, W1):
    return pl.pallas_call(
        _b_body,
        grid=(ROWS // RBLK,),
        in_specs=[
            pl.BlockSpec((NC, RBLK, 1), lambda i: (0, i, 0)),
            pl.BlockSpec((RBLK, D), lambda i: (i, 0)),
            pl.BlockSpec((D, D), lambda i: (0, 0)),
        ],
        out_specs=[
            pl.BlockSpec((RBLK, D), lambda i: (i, 0)),
            pl.BlockSpec((RBLK, 1), lambda i: (i, 0)),
        ],
        out_shape=[
            jax.ShapeDtypeStruct((ROWS, D), jnp.float32),
            jax.ShapeDtypeStruct((ROWS, 1), jnp.float32),
        ],
    )(degp, x_p, W1)


def _d_body(sp_ref, y1_ref, dinv_ref, b1_ref, w2_ref, y2_ref):
    stot = sp_ref[0] + sp_ref[1] + y1_ref[...]
    dinv = dinv_ref[...]
    h = jnp.maximum(stot * dinv + b1_ref[...], 0.0)
    y2_ref[...] = jnp.dot(h, w2_ref[...],
                          preferred_element_type=jnp.float32) * dinv


def _d_call(s1, y1, dinv, b1, W2):
    return pl.pallas_call(
        _d_body,
        grid=(ROWS // RBLK,),
        in_specs=[
            pl.BlockSpec((NC, RBLK, D), lambda i: (0, i, 0)),
            pl.BlockSpec((RBLK, D), lambda i: (i, 0)),
            pl.BlockSpec((RBLK, 1), lambda i: (i, 0)),
            pl.BlockSpec((1, D), lambda i: (0, 0)),
            pl.BlockSpec((D, D), lambda i: (0, 0)),
        ],
        out_specs=pl.BlockSpec((RBLK, D), lambda i: (i, 0)),
        out_shape=jax.ShapeDtypeStruct((ROWS, D), jnp.float32),
    )(s1, y1, dinv, b1, W2)


def _e_body(sp_ref, y2_ref, dinv_ref, b2_ref, batch_ref, wfc_ref, bfc_ref,
            out_ref, sums, counts):
    i = pl.program_id(0)

    @pl.when(i == 0)
    def _():
        sums[...] = jnp.zeros_like(sums)
        counts[...] = jnp.zeros_like(counts)

    stot = sp_ref[0] + sp_ref[1] + y2_ref[...]
    h = jnp.maximum(stot * dinv_ref[...] + b2_ref[...], 0.0)
    ids = batch_ref[0]
    mask = (lax.broadcasted_iota(jnp.int32, (NGRAPHS, RBLK), 0)
            == ids).astype(jnp.float32)
    sums[...] += jnp.dot(mask, h, preferred_element_type=jnp.float32)
    counts[...] += jnp.sum(mask, axis=1, keepdims=True)

    @pl.when(i == ROWS // RBLK - 1)
    def _():
        g = sums[...] / jnp.maximum(counts[...], 1.0)
        z = jnp.dot(g, wfc_ref[...],
                    preferred_element_type=jnp.float32) + bfc_ref[...]
        out_ref[...] = jax.nn.sigmoid(z)


def _e_call(s2, y2, dinv, b2, batch_p, Wfc, bfc):
    return pl.pallas_call(
        _e_body,
        grid=(ROWS // RBLK,),
        in_specs=[
            pl.BlockSpec((NC, RBLK, D), lambda i: (0, i, 0)),
            pl.BlockSpec((RBLK, D), lambda i: (i, 0)),
            pl.BlockSpec((RBLK, 1), lambda i: (i, 0)),
            pl.BlockSpec((1, D), lambda i: (0, 0)),
            pl.BlockSpec((1, 1, RBLK), lambda i: (i, 0, 0)),
            pl.BlockSpec((D, 1), lambda i: (0, 0)),
            pl.BlockSpec((1, 1), lambda i: (0, 0)),
        ],
        out_specs=pl.BlockSpec((NGRAPHS, 1), lambda i: (0, 0)),
        out_shape=jax.ShapeDtypeStruct((NGRAPHS, 1), jnp.float32),
        scratch_shapes=[
            pltpu.VMEM((NGRAPHS, D), jnp.float32),
            pltpu.VMEM((NGRAPHS, 1), jnp.float32),
        ],
    )(s2, y2, dinv, b2, batch_p, Wfc, bfc)


def kernel(x, edge_index, batch, W1, b1, W2, b2, Wfc, bfc):
    npad = EPAD - NEDGES
    spread = jnp.arange(npad, dtype=jnp.int32) % 16
    src_p = jnp.concatenate([edge_index[0], spread])
    dst_p = jnp.concatenate([edge_index[1], NNODES + spread])
    src3 = src_p.reshape(NW, NCHUNK, CHUNK)
    dst3 = dst_p.reshape(NW, NCHUNK, CHUNK)
    x_p = jnp.concatenate([x, jnp.zeros((ROWS - NNODES, D), x.dtype)])
    batch_p = jnp.concatenate(
        [batch, jnp.full((ROWS - NNODES,), NGRAPHS, jnp.int32)]
    ).reshape(ROWS // RBLK, 1, RBLK)
    zerosA = jnp.zeros((DEGR * 16 // NS, 16), jnp.float32)
    onesA = jnp.ones((CHUNK, 16), jnp.float32)
    zerosC = jnp.zeros((RPT, D), jnp.float32)

    deg_parts = _deg_call(dst3, onesA, zerosA)
    degp = deg_parts.reshape(NC, DEGR * 16, 16)[:, :ROWS, :1]
    y1, dinv = _b_call(degp, x_p, W1)
    s1 = _scat_call(y1, src3, dst3, zerosC)
    y2 = _d_call(s1, y1, dinv, b1.reshape(1, D), W2)
    s2 = _scat_call(y2, src3, dst3, zerosC)
    out = _e_call(s2, y2, dinv, b2.reshape(1, D), batch_p, Wfc,
                  bfc.reshape(1, 1))
    return out


# speedup vs baseline: 28.7888x; 1.0007x over previous
import functools

import jax
import jax.numpy as jnp
from jax import lax
from jax.experimental import pallas as pl
from jax.experimental.pallas import tpu as pltpu
from jax.experimental.pallas import tpu_sc as plsc

NNODES = 10000
NEDGES = 320000
D = 128
NGRAPHS = 256

NC, NS = 2, 16
NW = NC * NS
ROWS = 10016
RPT = ROWS // NS
EPAD = 327680
CHUNK = 128
NCHUNK = EPAD // (NW * CHUNK)
DCHUNK = 128
NDCHUNK = EPAD // (NW * DCHUNK)
DEGR = 640
RBLK = 2504

_mesh = plsc.VectorSubcoreMesh(
    core_axis_name="c", subcore_axis_name="s", num_cores=NC, num_subcores=NS)


def _deg_body(dst_hbm, ones_hbm, zero_hbm, out_hbm, dstix, ones_v, shared):
    c = lax.axis_index("c")
    s = lax.axis_index("s")
    wid = s * NC + c
    seg = DEGR * 16 // NS
    pltpu.sync_copy(dst_hbm.at[wid], dstix)
    pltpu.sync_copy(ones_hbm, ones_v)
    pltpu.sync_copy(zero_hbm, shared.at[pl.ds(s * seg, seg)])
    plsc.subcore_barrier()

    def chunk(j, carry):
        pltpu.sync_copy(ones_v, shared.at[dstix.at[j]], add=True)
        return carry

    lax.fori_loop(0, NDCHUNK, chunk, 0)
    plsc.subcore_barrier()
    pltpu.sync_copy(shared.at[pl.ds(s * seg, seg)], out_hbm.at[c, s])


def _deg_call(dst3, onesA, zerosA):
    f = pl.kernel(
        _deg_body,
        out_type=jax.ShapeDtypeStruct((NC, NS, DEGR * 16 // NS, 16),
                                      jnp.float32),
        mesh=_mesh,
        scratch_types=[
            pltpu.VMEM((NDCHUNK, DCHUNK), jnp.int32),
            pltpu.VMEM((DCHUNK, 16), jnp.float32),
            pltpu.VMEM_SHARED((DEGR * 16, 16), jnp.float32),
        ],
    )
    return f(dst3, onesA, zerosA)


def _scat_body(y_hbm, src_hbm, dst_hbm, zero_hbm, out_hbm,
               srcix, dstix, rowbuf, accum, *sems):
    c = lax.axis_index("c")
    s = lax.axis_index("s")
    wid = s * NC + c
    rsems = sems[0:2]
    isems = sems[2:6]
    pltpu.sync_copy(zero_hbm, accum.at[pl.ds(s * RPT, RPT)])
    plsc.subcore_barrier()

    def issue_idx(j, k):
        pltpu.async_copy(src_hbm.at[wid, j], srcix.at[k], isems[k])
        pltpu.async_copy(dst_hbm.at[wid, j], dstix.at[k], isems[k])

    def wait_idx(k):
        pltpu.make_async_copy(src_hbm.at[0, 0], srcix.at[k], isems[k]).wait()
        pltpu.make_async_copy(dst_hbm.at[0, 0], dstix.at[k], isems[k]).wait()

    for k in range(4):
        issue_idx(k, k)
    for b in range(2):
        wait_idx(b)
        pltpu.async_copy(y_hbm.at[srcix.at[b]], rowbuf.at[b], rsems[b])

    def outer(t, carry):
        for b in range(4):
            j = 4 * t + b
            rb = b % 2
            pltpu.make_async_copy(
                y_hbm.at[srcix.at[0]], rowbuf.at[rb], rsems[rb]).wait()
            pltpu.sync_copy(rowbuf.at[rb], accum.at[dstix.at[b]], add=True)
            issue_idx(jnp.minimum(j + 4, NCHUNK - 1), b)
            k2 = (b + 2) % 4
            wait_idx(k2)
            pltpu.async_copy(y_hbm.at[srcix.at[k2]], rowbuf.at[rb],
                             rsems[rb])
        return carry

    lax.fori_loop(0, NCHUNK // 4, outer, 0)
    for k in (2, 3):
        wait_idx(k)
    for b in range(2):
        pltpu.make_async_copy(
            y_hbm.at[srcix.at[0]], rowbuf.at[b], rsems[b]).wait()
    plsc.subcore_barrier()
    pltpu.sync_copy(accum.at[pl.ds(s * RPT, RPT)], out_hbm.at[c, s])


def _scat_call(y, src3, dst3, zerosC):
    f = pl.kernel(
        _scat_body,
        out_type=jax.ShapeDtypeStruct((NC, NS, RPT, D), jnp.float32),
        mesh=_mesh,
        scratch_types=[
            pltpu.VMEM((4, CHUNK), jnp.int32),
            pltpu.VMEM((4, CHUNK), jnp.int32),
            pltpu.VMEM((2, CHUNK, D), jnp.float32),
            pltpu.VMEM_SHARED((ROWS, D), jnp.float32),
        ] + [pltpu.SemaphoreType.DMA] * 6,
    )
    return f(y, src3, dst3, zerosC).reshape(NC, ROWS, D)


def _b_body(deg_ref, x_ref, w1_ref, y_ref, dinv_ref):
    deg = deg_ref[0] + deg_ref[1] + 1.0
    dinv = lax.rsqrt(deg)
    xw = jnp.dot(x_ref[...], w1_ref[...], preferred_element_type=jnp.float32)
    y_ref[...] = xw * dinv
    dinv_ref[...] = dinv


def _b_call(degp, x_p, W1):
    return pl.pallas_call(
        _b_body,
        grid=(ROWS // RBLK,),
        in_specs=[
            pl.BlockSpec((NC, RBLK, 1), lambda i: (0, i, 0)),
            pl.BlockSpec((RBLK, D), lambda i: (i, 0)),
            pl.BlockSpec((D, D), lambda i: (0, 0)),
        ],
        out_specs=[
            pl.BlockSpec((RBLK, D), lambda i: (i, 0)),
            pl.BlockSpec((RBLK, 1), lambda i: (i, 0)),
        ],
        out_shape=[
            jax.ShapeDtypeStruct((ROWS, D), jnp.float32),
            jax.ShapeDtypeStruct((ROWS, 1), jnp.float32),
        ],
    )(degp, x_p, W1)


def _d_body(sp_ref, y1_ref, dinv_ref, b1_ref, w2_ref, y2_ref):
    stot = sp_ref[0] + sp_ref[1] + y1_ref[...]
    dinv = dinv_ref[...]
    h = jnp.maximum(stot * dinv + b1_ref[...], 0.0)
    y2_ref[...] = jnp.dot(h, w2_ref[...],
                          preferred_element_type=jnp.float32) * dinv


def _d_call(s1, y1, dinv, b1, W2):
    return pl.pallas_call(
        _d_body,
        grid=(ROWS // RBLK,),
        in_specs=[
            pl.BlockSpec((NC, RBLK, D), lambda i: (0, i, 0)),
            pl.BlockSpec((RBLK, D), lambda i: (i, 0)),
            pl.BlockSpec((RBLK, 1), lambda i: (i, 0)),
            pl.BlockSpec((1, D), lambda i: (0, 0)),
            pl.BlockSpec((D, D), lambda i: (0, 0)),
        ],
        out_specs=pl.BlockSpec((RBLK, D), lambda i: (i, 0)),
        out_shape=jax.ShapeDtypeStruct((ROWS, D), jnp.float32),
    )(s1, y1, dinv, b1, W2)


def _e_body(sp_ref, y2_ref, dinv_ref, b2_ref, batch_ref, wfc_ref, bfc_ref,
            out_ref, sums, counts):
    i = pl.program_id(0)

    @pl.when(i == 0)
    def _():
        sums[...] = jnp.zeros_like(sums)
        counts[...] = jnp.zeros_like(counts)

    stot = sp_ref[0] + sp_ref[1] + y2_ref[...]
    h = jnp.maximum(stot * dinv_ref[...] + b2_ref[...], 0.0)
    ids = batch_ref[0]
    mask = (lax.broadcasted_iota(jnp.int32, (NGRAPHS, RBLK), 0)
            == ids).astype(jnp.float32)
    sums[...] += jnp.dot(mask, h, preferred_element_type=jnp.float32)
    counts[...] += jnp.sum(mask, axis=1, keepdims=True)

    @pl.when(i == ROWS // RBLK - 1)
    def _():
        g = sums[...] / jnp.maximum(counts[...], 1.0)
        z = jnp.dot(g, wfc_ref[...],
                    preferred_element_type=jnp.float32) + bfc_ref[...]
        out_ref[...] = jax.nn.sigmoid(z)


def _e_call(s2, y2, dinv, b2, batch_p, Wfc, bfc):
    return pl.pallas_call(
        _e_body,
        grid=(ROWS // RBLK,),
        in_specs=[
            pl.BlockSpec((NC, RBLK, D), lambda i: (0, i, 0)),
            pl.BlockSpec((RBLK, D), lambda i: (i, 0)),
            pl.BlockSpec((RBLK, 1), lambda i: (i, 0)),
            pl.BlockSpec((1, D), lambda i: (0, 0)),
            pl.BlockSpec((1, 1, RBLK), lambda i: (i, 0, 0)),
            pl.BlockSpec((D, 1), lambda i: (0, 0)),
            pl.BlockSpec((1, 1), lambda i: (0, 0)),
        ],
        out_specs=pl.BlockSpec((NGRAPHS, 1), lambda i: (0, 0)),
        out_shape=jax.ShapeDtypeStruct((NGRAPHS, 1), jnp.float32),
        scratch_shapes=[
            pltpu.VMEM((NGRAPHS, D), jnp.float32),
            pltpu.VMEM((NGRAPHS, 1), jnp.float32),
        ],
    )(s2, y2, dinv, b2, batch_p, Wfc, bfc)


def kernel(x, edge_index, batch, W1, b1, W2, b2, Wfc, bfc):
    npad = EPAD - NEDGES
    spread = jnp.arange(npad, dtype=jnp.int32) % 16
    src_p = jnp.concatenate([edge_index[0], spread])
    dst_p = jnp.concatenate([edge_index[1], NNODES + spread])
    src3 = src_p.reshape(NW, NCHUNK, CHUNK)
    dst3 = dst_p.reshape(NW, NCHUNK, CHUNK)
    dst3d = dst_p.reshape(NW, NDCHUNK, DCHUNK)
    x_p = jnp.concatenate([x, jnp.zeros((ROWS - NNODES, D), x.dtype)])
    batch_p = jnp.concatenate(
        [batch, jnp.full((ROWS - NNODES,), NGRAPHS, jnp.int32)]
    ).reshape(ROWS // RBLK, 1, RBLK)
    zerosA = jnp.zeros((DEGR * 16 // NS, 16), jnp.float32)
    onesA = jnp.ones((DCHUNK, 16), jnp.float32)
    zerosC = jnp.zeros((RPT, D), jnp.float32)

    deg_parts = _deg_call(dst3d, onesA, zerosA)
    degp = deg_parts.reshape(NC, DEGR * 16, 16)[:, :ROWS, :1]
    y1, dinv = _b_call(degp, x_p, W1)
    s1 = _scat_call(y1, src3, dst3, zerosC)
    y2 = _d_call(s1, y1, dinv, b1.reshape(1, D), W2)
    s2 = _scat_call(y2, src3, dst3, zerosC)
    out = _e_call(s2, y2, dinv, b2.reshape(1, D), batch_p, Wfc,
                  bfc.reshape(1, 1))
    return out
